# speedup vs baseline: 1.8592x; 1.8592x over previous
import functools

import jax
import jax.numpy as jnp
from jax import lax
from jax.experimental import pallas as pl
from jax.experimental.pallas import tpu as pltpu
from jax.experimental.pallas import tpu_sc as plsc

N = 10000
D = 128
DR = 16
DH = 128
DFF = 512
E = 320000

NC = 2
NS = 16
NW = NC * NS
CHUNK = 128
E_PER_TILE = 10240
E_PAD = NW * E_PER_TILE
N_CHUNKS = E_PER_TILE // CHUNK
N_PAD = 10240
ROWS_PER_SUBCORE = N_PAD // NS


def _pre_body(x_ref, g_ref, b_ref, w_ref, bias_ref, o_ref):
    x = x_ref[...]
    mu = jnp.mean(x, axis=-1, keepdims=True)
    xc = x - mu
    var = jnp.mean(xc * xc, axis=-1, keepdims=True)
    xn = xc * lax.rsqrt(var + 1e-5) * g_ref[...] + b_ref[...]
    o_ref[...] = (jnp.dot(xn, w_ref[...], preferred_element_type=jnp.float32)
                  + bias_ref[...])


def _h_pre(x, ln1_g, ln1_b, W_pre, b_pre):
    bn = 1000
    return pl.pallas_call(
        _pre_body,
        grid=(N // bn,),
        in_specs=[
            pl.BlockSpec((bn, D), lambda i: (i, 0)),
            pl.BlockSpec((1, D), lambda i: (0, 0)),
            pl.BlockSpec((1, D), lambda i: (0, 0)),
            pl.BlockSpec((D, DH), lambda i: (0, 0)),
            pl.BlockSpec((1, DH), lambda i: (0, 0)),
        ],
        out_specs=pl.BlockSpec((bn, DH), lambda i: (i, 0)),
        out_shape=jax.ShapeDtypeStruct((N, DH), jnp.float32),
    )(x, ln1_g.reshape(1, D), ln1_b.reshape(1, D), W_pre, b_pre.reshape(1, DH))


def _filt_body(rb_ref, w_ref, bias_ref, o_ref):
    o_ref[...] = (jnp.dot(rb_ref[...], w_ref[...],
                          preferred_element_type=jnp.float32) + bias_ref[...])


def _filt(rb_pad, W_rf, b_rf):
    be = 8192
    return pl.pallas_call(
        _filt_body,
        grid=(E_PAD // be,),
        in_specs=[
            pl.BlockSpec((be, DR), lambda i: (i, 0)),
            pl.BlockSpec((DR, DH), lambda i: (0, 0)),
            pl.BlockSpec((1, DH), lambda i: (0, 0)),
        ],
        out_specs=pl.BlockSpec((be, DH), lambda i: (i, 0)),
        out_shape=jax.ShapeDtypeStruct((E_PAD, DH), jnp.float32),
    )(rb_pad, W_rf, b_rf.reshape(1, DH))


def _sc_conv(h_pre, src_p, dst_p, filt):
    mesh = plsc.VectorSubcoreMesh(core_axis_name="c", subcore_axis_name="s")

    @functools.partial(
        pl.kernel,
        mesh=mesh,
        out_type=jax.ShapeDtypeStruct((NC, N_PAD, DH), jnp.float32),
        scratch_types=[
            pltpu.VMEM((CHUNK,), jnp.int32),
            pltpu.VMEM((CHUNK,), jnp.int32),
            pltpu.VMEM((CHUNK, DH), jnp.float32),
            pltpu.VMEM((CHUNK, DH), jnp.float32),
            pltpu.VMEM_SHARED((N_PAD, DH), jnp.float32),
            pltpu.SemaphoreType.DMA,
        ],
    )
    def conv(h_hbm, src_hbm, dst_hbm, filt_hbm, out_hbm,
             sidx_v, didx_v, gath_v, filt_v, agg_sh, sem):
        cid = lax.axis_index("c")
        sid = lax.axis_index("s")
        wid = sid * NC + cid

        @pl.loop(0, CHUNK)
        def _zrow(r):
            for c in range(DH // 16):
                gath_v[r, pl.ds(c * 16, 16)] = jnp.zeros((16,), jnp.float32)

        row0 = sid * ROWS_PER_SUBCORE

        @pl.loop(0, ROWS_PER_SUBCORE // CHUNK)
        def _zcopy(j):
            pltpu.sync_copy(gath_v, agg_sh.at[pl.ds(row0 + j * CHUNK, CHUNK)])

        plsc.subcore_barrier()

        @pl.loop(0, N_CHUNKS)
        def _step(g):
            base = wid * E_PER_TILE + g * CHUNK
            pltpu.sync_copy(src_hbm.at[pl.ds(base, CHUNK)], sidx_v)
            pltpu.sync_copy(dst_hbm.at[pl.ds(base, CHUNK)], didx_v)
            pltpu.sync_copy(filt_hbm.at[pl.ds(base, CHUNK)], filt_v)
            pltpu.async_copy(h_hbm.at[sidx_v], gath_v, sem).wait()

            @pl.loop(0, CHUNK)
            def _mul(r):
                for c in range(DH // 16):
                    sl = pl.ds(c * 16, 16)
                    gath_v[r, sl] = gath_v[r, sl] * filt_v[r, sl]

            pltpu.sync_copy(gath_v, agg_sh.at[didx_v], add=True)

        plsc.subcore_barrier()
        pltpu.sync_copy(agg_sh.at[pl.ds(row0, ROWS_PER_SUBCORE)],
                        out_hbm.at[cid, pl.ds(row0, ROWS_PER_SUBCORE)])

    return conv(h_pre, src_p, dst_p, filt)


def _final_body(p_ref, x_ref, wpost_ref, bpost_ref, g_ref, b_ref,
                w1_ref, b1_ref, w2_ref, b2_ref, o_ref):
    agg = p_ref[0] + p_ref[1]
    h = (jnp.dot(agg, wpost_ref[...], preferred_element_type=jnp.float32)
         + bpost_ref[...])
    h = h * jax.nn.sigmoid(h) + x_ref[...]
    mu = jnp.mean(h, axis=-1, keepdims=True)
    hc = h - mu
    var = jnp.mean(hc * hc, axis=-1, keepdims=True)
    f = hc * lax.rsqrt(var + 1e-5) * g_ref[...] + b_ref[...]
    f1 = (jnp.dot(f, w1_ref[...], preferred_element_type=jnp.float32)
          + b1_ref[...])
    f1 = f1 * jax.nn.sigmoid(f1)
    o_ref[...] = (jnp.dot(f1, w2_ref[...], preferred_element_type=jnp.float32)
                  + b2_ref[...] + h)


def _final(partials, x, W_post, b_post, ln2_g, ln2_b, W_ff1, b_ff1,
           W_ff2, b_ff2):
    bn = 1000
    return pl.pallas_call(
        _final_body,
        grid=(N // bn,),
        in_specs=[
            pl.BlockSpec((NC, bn, DH), lambda i: (0, i, 0)),
            pl.BlockSpec((bn, D), lambda i: (i, 0)),
            pl.BlockSpec((DH, D), lambda i: (0, 0)),
            pl.BlockSpec((1, D), lambda i: (0, 0)),
            pl.BlockSpec((1, D), lambda i: (0, 0)),
            pl.BlockSpec((1, D), lambda i: (0, 0)),
            pl.BlockSpec((D, DFF), lambda i: (0, 0)),
            pl.BlockSpec((1, DFF), lambda i: (0, 0)),
            pl.BlockSpec((DFF, D), lambda i: (0, 0)),
            pl.BlockSpec((1, D), lambda i: (0, 0)),
        ],
        out_specs=pl.BlockSpec((bn, D), lambda i: (i, 0)),
        out_shape=jax.ShapeDtypeStruct((N, D), jnp.float32),
    )(partials, x, W_post, b_post.reshape(1, D), ln2_g.reshape(1, D),
      ln2_b.reshape(1, D), W_ff1, b_ff1.reshape(1, DFF), W_ff2,
      b_ff2.reshape(1, D))


def kernel(x, edge_index, radial_basis, W_pre, b_pre, W_rf, b_rf,
           W_post, b_post, ln1_g, ln1_b, ln2_g, ln2_b,
           W_ff1, b_ff1, W_ff2, b_ff2):
    pad = E_PAD - E
    src_p = jnp.concatenate(
        [edge_index[0].astype(jnp.int32), jnp.zeros((pad,), jnp.int32)])
    dst_p = jnp.concatenate(
        [edge_index[1].astype(jnp.int32), jnp.full((pad,), N, jnp.int32)])
    rb_p = jnp.concatenate(
        [radial_basis, jnp.zeros((pad, DR), jnp.float32)])

    h_pre = _h_pre(x, ln1_g, ln1_b, W_pre, b_pre)
    filt = _filt(rb_p, W_rf, b_rf)
    partials = _sc_conv(h_pre, src_p, dst_p, filt)
    return _final(partials, x, W_post, b_post, ln2_g, ln2_b,
                  W_ff1, b_ff1, W_ff2, b_ff2)


# speedup vs baseline: 2.0879x; 1.1230x over previous
import dataclasses
import functools

import jax
import jax.numpy as jnp
import numpy as np
from jax import lax
from jax.experimental import pallas as pl
from jax.experimental.pallas import tpu as pltpu
from jax.experimental.pallas import tpu_sc as plsc

N = 10000
D = 128
DR = 16
DH = 128
DFF = 512
E = 320000

NC = 2
NS = 16
NW = NC * NS
CHUNK = 128
E_PER_TILE = 10240
E_PAD = NW * E_PER_TILE
N_CHUNKS = E_PER_TILE // CHUNK
N_ACC = 10112
ROWS_PER_SUBCORE = N_ACC // NS

_LCOLS = np.concatenate([np.arange(16) + 32 * c for c in range(4)])
_HCOLS = _LCOLS + 16
DHW = DH // 2


def _pack16(lo_f32, hi_f32):
    lo = lax.convert_element_type(
        lax.bitcast_convert_type(lo_f32.astype(jnp.bfloat16), jnp.uint16),
        jnp.uint32)
    hi = lax.convert_element_type(
        lax.bitcast_convert_type(hi_f32.astype(jnp.bfloat16), jnp.uint16),
        jnp.uint32)
    return lax.bitcast_convert_type(lo | (hi << 16), jnp.int32)


def _pre_body(x_ref, g_ref, b_ref, w_ref, bias_ref, o_ref):
    x = x_ref[...]
    mu = jnp.mean(x, axis=-1, keepdims=True)
    xc = x - mu
    var = jnp.mean(xc * xc, axis=-1, keepdims=True)
    xn = xc * lax.rsqrt(var + 1e-5) * g_ref[...] + b_ref[...]
    o_ref[...] = (jnp.dot(xn, w_ref[...], preferred_element_type=jnp.float32)
                  + bias_ref[...])


def _h_pre(x, ln1_g, ln1_b, W_pre, b_pre):
    bn = 1000
    return pl.pallas_call(
        _pre_body,
        grid=(N // bn,),
        in_specs=[
            pl.BlockSpec((bn, D), lambda i: (i, 0)),
            pl.BlockSpec((1, D), lambda i: (0, 0)),
            pl.BlockSpec((1, D), lambda i: (0, 0)),
            pl.BlockSpec((D, DH), lambda i: (0, 0)),
            pl.BlockSpec((1, DH), lambda i: (0, 0)),
        ],
        out_specs=pl.BlockSpec((bn, DH), lambda i: (i, 0)),
        out_shape=jax.ShapeDtypeStruct((N, DH), jnp.float32),
    )(x, ln1_g.reshape(1, D), ln1_b.reshape(1, D), W_pre,
      b_pre.reshape(1, DH))


_FILT_BE = 8192


def _filt_body(rb_ref, wl_ref, wh_ref, bl_ref, bh_ref, o_ref):
    i = pl.program_id(0)
    fl = (jnp.dot(rb_ref[...], wl_ref[...],
                  preferred_element_type=jnp.float32) + bl_ref[...])
    fh = (jnp.dot(rb_ref[...], wh_ref[...],
                  preferred_element_type=jnp.float32) + bh_ref[...])
    rows = lax.broadcasted_iota(jnp.int32, fl.shape, 0) + i * _FILT_BE
    fl = jnp.where(rows < E, fl, 0.0)
    fh = jnp.where(rows < E, fh, 0.0)
    o_ref[...] = _pack16(fl, fh)


def _filt(rb_pad, W_l, W_h, b_l, b_h):
    be = _FILT_BE
    return pl.pallas_call(
        _filt_body,
        grid=(E_PAD // be,),
        in_specs=[
            pl.BlockSpec((be, DR), lambda i: (i, 0)),
            pl.BlockSpec((DR, DHW), lambda i: (0, 0)),
            pl.BlockSpec((DR, DHW), lambda i: (0, 0)),
            pl.BlockSpec((1, DHW), lambda i: (0, 0)),
            pl.BlockSpec((1, DHW), lambda i: (0, 0)),
        ],
        out_specs=pl.BlockSpec((be, DHW), lambda i: (i, 0)),
        out_shape=jax.ShapeDtypeStruct((E_PAD, DHW), jnp.int32),
    )(rb_pad, W_l, W_h, b_l, b_h)


def _sc_conv(h_pre, src_r, dst_r, filt):
    mesh = plsc.VectorSubcoreMesh(core_axis_name="c", subcore_axis_name="s")
    cp = pltpu.CompilerParams()
    if "needs_layout_passes" in pltpu.CompilerParams.__dataclass_fields__:
        cp = dataclasses.replace(cp, needs_layout_passes=False)

    @functools.partial(
        pl.kernel,
        mesh=mesh,
        compiler_params=cp,
        out_type=jax.ShapeDtypeStruct((NC, N_ACC, DH), jnp.float32),
        scratch_types=[
            pltpu.VMEM((2, CHUNK), jnp.int32),
            pltpu.VMEM((2, CHUNK), jnp.int32),
            pltpu.VMEM((2, CHUNK, DH), jnp.float32),
            pltpu.VMEM((2, CHUNK // 2, DH), jnp.int32),
            pltpu.VMEM_SHARED((N_ACC, DH), jnp.float32),
            pltpu.SemaphoreType.DMA,
            pltpu.SemaphoreType.DMA,
            pltpu.SemaphoreType.DMA,
            pltpu.SemaphoreType.DMA,
            pltpu.SemaphoreType.DMA,
            pltpu.SemaphoreType.DMA,
        ],
    )
    def conv(h_hbm, src_hbm, dst_hbm, filt_hbm, out_hbm,
             sidx_v, didx_v, gath_v, filt_v, agg_sh,
             si0, si1, sg0, sg1, sf0, sf1):
        sis, sgs, sfs = (si0, si1), (sg0, sg1), (sf0, sf1)
        cid = lax.axis_index("c")
        sid = lax.axis_index("s")
        wid = sid * NC + cid
        ebase = wid * E_PER_TILE

        def idx_start(gg, s):
            eo = pl.ds(ebase + gg * CHUNK, CHUNK)
            pltpu.async_copy(src_hbm.at[eo], sidx_v.at[s], sis[s])
            pltpu.async_copy(dst_hbm.at[eo], didx_v.at[s], sis[s])

        def idx_wait(gg, s):
            eo = pl.ds(ebase + gg * CHUNK, CHUNK)
            pltpu.make_async_copy(src_hbm.at[eo], sidx_v.at[s],
                                  sis[s]).wait()
            pltpu.make_async_copy(dst_hbm.at[eo], didx_v.at[s],
                                  sis[s]).wait()

        def gf_start(gg, s):
            pltpu.async_copy(h_hbm.at[sidx_v.at[s]], gath_v.at[s], sgs[s])
            fo = pl.multiple_of(
                wid * (E_PER_TILE // 2) + gg * (CHUNK // 2), CHUNK // 2)
            pltpu.async_copy(filt_hbm.at[pl.ds(fo, CHUNK // 2)],
                             filt_v.at[s], sfs[s])

        def gf_wait(gg, s):
            pltpu.make_async_copy(h_hbm.at[sidx_v.at[s]], gath_v.at[s],
                                  sgs[s]).wait()
            fo = pl.multiple_of(
                wid * (E_PER_TILE // 2) + gg * (CHUNK // 2), CHUNK // 2)
            pltpu.make_async_copy(filt_hbm.at[pl.ds(fo, CHUNK // 2)],
                                  filt_v.at[s], sfs[s]).wait()

        idx_start(0, 0)
        idx_start(1, 1)

        @pl.loop(0, CHUNK)
        def _zrow(r):
            for c in range(DH // 16):
                gath_v[0, r, pl.ds(c * 16, 16)] = jnp.zeros((16,),
                                                            jnp.float32)

        row0 = sid * ROWS_PER_SUBCORE

        for j in range(4):
            pltpu.sync_copy(gath_v.at[0],
                            agg_sh.at[pl.ds(row0 + j * CHUNK, CHUNK)])
        pltpu.sync_copy(gath_v.at[0, pl.ds(0, ROWS_PER_SUBCORE - 4 * CHUNK)],
                        agg_sh.at[pl.ds(row0 + 4 * CHUNK,
                                        ROWS_PER_SUBCORE - 4 * CHUNK)])

        idx_wait(0, 0)
        gf_start(0, 0)
        plsc.subcore_barrier()

        @pl.loop(0, N_CHUNKS, step=2)
        def _step(g):
            for b in range(2):
                gg = g + b
                gf_wait(gg, b)

                @pl.when(gg + 1 < N_CHUNKS)
                def _nxt():
                    idx_wait(gg + 1, 1 - b)
                    gf_start(gg + 1, 1 - b)

                @pl.loop(0, CHUNK // 2)
                def _mul(rp):
                    for half in range(2):
                        r = rp * 2 + half
                        for c in range(DHW // 16):
                            fb = plsc.bitcast(
                                filt_v[b, rp,
                                       pl.ds(half * DHW + c * 16, 16)],
                                jnp.bfloat16)
                            lo, hi = plsc.unpack(
                                fb, format=plsc.PackFormat.INTERLEAVED)
                            sl0 = pl.ds(c * 32, 16)
                            sl1 = pl.ds(c * 32 + 16, 16)
                            gath_v[b, r, sl0] = gath_v[b, r, sl0] * lo
                            gath_v[b, r, sl1] = gath_v[b, r, sl1] * hi

                pltpu.sync_copy(gath_v.at[b], agg_sh.at[didx_v.at[b]],
                                add=True)

                @pl.when(gg + 2 < N_CHUNKS)
                def _pref():
                    idx_start(gg + 2, b)

        plsc.subcore_barrier()
        pltpu.sync_copy(agg_sh.at[pl.ds(row0, ROWS_PER_SUBCORE)],
                        out_hbm.at[cid, pl.ds(row0, ROWS_PER_SUBCORE)])

    return conv(h_pre, src_r, dst_r, filt)


def _final_body(p_ref, x_ref, wpost_ref, bpost_ref, g_ref, b_ref,
                w1_ref, b1_ref, w2_ref, b2_ref, o_ref):
    agg = p_ref[0] + p_ref[1]
    h = (jnp.dot(agg, wpost_ref[...], preferred_element_type=jnp.float32)
         + bpost_ref[...])
    h = h * jax.nn.sigmoid(h) + x_ref[...]
    mu = jnp.mean(h, axis=-1, keepdims=True)
    hc = h - mu
    var = jnp.mean(hc * hc, axis=-1, keepdims=True)
    f = hc * lax.rsqrt(var + 1e-5) * g_ref[...] + b_ref[...]
    f1 = (jnp.dot(f, w1_ref[...], preferred_element_type=jnp.float32)
          + b1_ref[...])
    f1 = f1 * jax.nn.sigmoid(f1)
    o_ref[...] = (jnp.dot(f1, w2_ref[...], preferred_element_type=jnp.float32)
                  + b2_ref[...] + h)


def _final(partials, x, W_post, b_post, ln2_g, ln2_b, W_ff1, b_ff1,
           W_ff2, b_ff2):
    bn = 1000
    return pl.pallas_call(
        _final_body,
        grid=(N // bn,),
        in_specs=[
            pl.BlockSpec((NC, bn, DH), lambda i: (0, i, 0)),
            pl.BlockSpec((bn, D), lambda i: (i, 0)),
            pl.BlockSpec((DH, D), lambda i: (0, 0)),
            pl.BlockSpec((1, D), lambda i: (0, 0)),
            pl.BlockSpec((1, D), lambda i: (0, 0)),
            pl.BlockSpec((1, D), lambda i: (0, 0)),
            pl.BlockSpec((D, DFF), lambda i: (0, 0)),
            pl.BlockSpec((1, DFF), lambda i: (0, 0)),
            pl.BlockSpec((DFF, D), lambda i: (0, 0)),
            pl.BlockSpec((1, D), lambda i: (0, 0)),
        ],
        out_specs=pl.BlockSpec((bn, D), lambda i: (i, 0)),
        out_shape=jax.ShapeDtypeStruct((N, D), jnp.float32),
    )(partials, x, W_post, b_post.reshape(1, D), ln2_g.reshape(1, D),
      ln2_b.reshape(1, D), W_ff1, b_ff1.reshape(1, DFF), W_ff2,
      b_ff2.reshape(1, D))


def kernel(x, edge_index, radial_basis, W_pre, b_pre, W_rf, b_rf,
           W_post, b_post, ln1_g, ln1_b, ln2_g, ln2_b,
           W_ff1, b_ff1, W_ff2, b_ff2):
    pad = E_PAD - E
    src_p = jnp.concatenate(
        [edge_index[0].astype(jnp.int32), jnp.zeros((pad,), jnp.int32)])
    dst_p = jnp.concatenate(
        [edge_index[1].astype(jnp.int32), jnp.zeros((pad,), jnp.int32)])
    rb_p = jnp.concatenate(
        [radial_basis, jnp.zeros((pad, DR), jnp.float32)])

    lcols = jnp.asarray(_LCOLS)
    hcols = jnp.asarray(_HCOLS)
    h_pre = _h_pre(x, ln1_g, ln1_b, W_pre, b_pre)
    filt = _filt(rb_p, W_rf[:, lcols], W_rf[:, hcols],
                 b_rf[lcols].reshape(1, DHW), b_rf[hcols].reshape(1, DHW))
    partials = _sc_conv(h_pre, src_p, dst_p,
                        filt.reshape(E_PAD // 2, DH))
    return _final(partials, x, W_post, b_post, ln2_g, ln2_b,
                  W_ff1, b_ff1, W_ff2, b_ff2)


# speedup vs baseline: 2.1248x; 1.0177x over previous
import dataclasses
import functools

import jax
import jax.numpy as jnp
import numpy as np
from jax import lax
from jax.experimental import pallas as pl
from jax.experimental.pallas import tpu as pltpu
from jax.experimental.pallas import tpu_sc as plsc

N = 10000
D = 128
DR = 16
DH = 128
DFF = 512
E = 320000

NC = 2
NS = 16
NW = NC * NS
CHUNK = 128
E_PER_TILE = 10240
E_PAD = NW * E_PER_TILE
N_CHUNKS = E_PER_TILE // CHUNK
N_ACC = 10112
ROWS_PER_SUBCORE = N_ACC // NS

_LCOLS = np.concatenate([np.arange(16) + 32 * c for c in range(4)])
_HCOLS = _LCOLS + 16
DHW = DH // 2


def _pack16(lo_f32, hi_f32):
    lo = lax.convert_element_type(
        lax.bitcast_convert_type(lo_f32.astype(jnp.bfloat16), jnp.uint16),
        jnp.uint32)
    hi = lax.convert_element_type(
        lax.bitcast_convert_type(hi_f32.astype(jnp.bfloat16), jnp.uint16),
        jnp.uint32)
    return lax.bitcast_convert_type(lo | (hi << 16), jnp.int32)


def _pre_body(x_ref, g_ref, b_ref, w_ref, bias_ref, o_ref):
    x = x_ref[...]
    mu = jnp.mean(x, axis=-1, keepdims=True)
    xc = x - mu
    var = jnp.mean(xc * xc, axis=-1, keepdims=True)
    xn = xc * lax.rsqrt(var + 1e-5) * g_ref[...] + b_ref[...]
    o_ref[...] = (jnp.dot(xn, w_ref[...], preferred_element_type=jnp.float32)
                  + bias_ref[...])


def _h_pre(x, ln1_g, ln1_b, W_pre, b_pre):
    bn = 1000
    return pl.pallas_call(
        _pre_body,
        grid=(N // bn,),
        in_specs=[
            pl.BlockSpec((bn, D), lambda i: (i, 0)),
            pl.BlockSpec((1, D), lambda i: (0, 0)),
            pl.BlockSpec((1, D), lambda i: (0, 0)),
            pl.BlockSpec((D, DH), lambda i: (0, 0)),
            pl.BlockSpec((1, DH), lambda i: (0, 0)),
        ],
        out_specs=pl.BlockSpec((bn, DH), lambda i: (i, 0)),
        out_shape=jax.ShapeDtypeStruct((N, DH), jnp.float32),
    )(x, ln1_g.reshape(1, D), ln1_b.reshape(1, D), W_pre,
      b_pre.reshape(1, DH))


_FILT_BE = 8192


def _filt_body(rb_ref, wl_ref, wh_ref, bl_ref, bh_ref, o_ref):
    i = pl.program_id(0)
    fl = (jnp.dot(rb_ref[...], wl_ref[...],
                  preferred_element_type=jnp.float32) + bl_ref[...])
    fh = (jnp.dot(rb_ref[...], wh_ref[...],
                  preferred_element_type=jnp.float32) + bh_ref[...])
    rows = lax.broadcasted_iota(jnp.int32, fl.shape, 0) + i * _FILT_BE
    fl = jnp.where(rows < E, fl, 0.0)
    fh = jnp.where(rows < E, fh, 0.0)
    o_ref[...] = _pack16(fl, fh)


def _filt(rb_pad, W_l, W_h, b_l, b_h):
    be = _FILT_BE
    return pl.pallas_call(
        _filt_body,
        grid=(E_PAD // be,),
        in_specs=[
            pl.BlockSpec((be, DR), lambda i: (i, 0)),
            pl.BlockSpec((DR, DHW), lambda i: (0, 0)),
            pl.BlockSpec((DR, DHW), lambda i: (0, 0)),
            pl.BlockSpec((1, DHW), lambda i: (0, 0)),
            pl.BlockSpec((1, DHW), lambda i: (0, 0)),
        ],
        out_specs=pl.BlockSpec((be, DHW), lambda i: (i, 0)),
        out_shape=jax.ShapeDtypeStruct((E_PAD, DHW), jnp.int32),
    )(rb_pad, W_l, W_h, b_l, b_h)


def _sc_conv(h_pre, src_r, dst_r, filt):
    mesh = plsc.VectorSubcoreMesh(core_axis_name="c", subcore_axis_name="s")
    cp = pltpu.CompilerParams(use_tc_tiling_on_sc=True)
    if "needs_layout_passes" in pltpu.CompilerParams.__dataclass_fields__:
        cp = dataclasses.replace(cp, needs_layout_passes=False)

    @functools.partial(
        pl.kernel,
        mesh=mesh,
        compiler_params=cp,
        out_type=jax.ShapeDtypeStruct((NC, N_ACC, DH), jnp.float32),
        scratch_types=[
            pltpu.VMEM((2, CHUNK), jnp.int32),
            pltpu.VMEM((2, CHUNK), jnp.int32),
            pltpu.VMEM((2, CHUNK, DH), jnp.float32),
            pltpu.VMEM((2, CHUNK // 2, DH), jnp.int32),
            pltpu.VMEM_SHARED((N_ACC, DH), jnp.float32),
            pltpu.SemaphoreType.DMA,
            pltpu.SemaphoreType.DMA,
            pltpu.SemaphoreType.DMA,
            pltpu.SemaphoreType.DMA,
            pltpu.SemaphoreType.DMA,
            pltpu.SemaphoreType.DMA,
        ],
    )
    def conv(h_hbm, src_hbm, dst_hbm, filt_hbm, out_hbm,
             sidx_v, didx_v, gath_v, filt_v, agg_sh,
             si0, si1, sg0, sg1, sf0, sf1):
        sis, sgs, sfs = (si0, si1), (sg0, sg1), (sf0, sf1)
        cid = lax.axis_index("c")
        sid = lax.axis_index("s")
        wid = sid * NC + cid
        ebase = wid * E_PER_TILE

        def idx_start(gg, s):
            eo = pl.ds(ebase + gg * CHUNK, CHUNK)
            pltpu.async_copy(src_hbm.at[eo], sidx_v.at[s], sis[s])
            pltpu.async_copy(dst_hbm.at[eo], didx_v.at[s], sis[s])

        def idx_wait(gg, s):
            eo = pl.ds(ebase + gg * CHUNK, CHUNK)
            pltpu.make_async_copy(src_hbm.at[eo], sidx_v.at[s],
                                  sis[s]).wait()
            pltpu.make_async_copy(dst_hbm.at[eo], didx_v.at[s],
                                  sis[s]).wait()

        def gf_start(gg, s):
            pltpu.async_copy(h_hbm.at[sidx_v.at[s]], gath_v.at[s], sgs[s])
            fo = pl.multiple_of(
                wid * (E_PER_TILE // 2) + gg * (CHUNK // 2), CHUNK // 2)
            pltpu.async_copy(filt_hbm.at[pl.ds(fo, CHUNK // 2)],
                             filt_v.at[s], sfs[s])

        def gf_wait(gg, s):
            pltpu.make_async_copy(h_hbm.at[sidx_v.at[s]], gath_v.at[s],
                                  sgs[s]).wait()
            fo = pl.multiple_of(
                wid * (E_PER_TILE // 2) + gg * (CHUNK // 2), CHUNK // 2)
            pltpu.make_async_copy(filt_hbm.at[pl.ds(fo, CHUNK // 2)],
                                  filt_v.at[s], sfs[s]).wait()

        idx_start(0, 0)
        idx_start(1, 1)

        @pl.loop(0, CHUNK)
        def _zrow(r):
            for c in range(DH // 16):
                gath_v[0, r, pl.ds(c * 16, 16)] = jnp.zeros((16,),
                                                            jnp.float32)

        row0 = sid * ROWS_PER_SUBCORE

        for j in range(4):
            pltpu.sync_copy(gath_v.at[0],
                            agg_sh.at[pl.ds(row0 + j * CHUNK, CHUNK)])
        pltpu.sync_copy(gath_v.at[0, pl.ds(0, ROWS_PER_SUBCORE - 4 * CHUNK)],
                        agg_sh.at[pl.ds(row0 + 4 * CHUNK,
                                        ROWS_PER_SUBCORE - 4 * CHUNK)])

        idx_wait(0, 0)
        gf_start(0, 0)
        plsc.subcore_barrier()

        @pl.loop(0, N_CHUNKS, step=2)
        def _step(g):
            for b in range(2):
                gg = g + b
                gf_wait(gg, b)

                @pl.when(gg + 1 < N_CHUNKS)
                def _nxt():
                    idx_wait(gg + 1, 1 - b)
                    gf_start(gg + 1, 1 - b)

                @pl.loop(0, CHUNK // 2)
                def _mul(rp):
                    for half in range(2):
                        r = rp * 2 + half
                        for c in range(DHW // 16):
                            fb = plsc.bitcast(
                                filt_v[b, rp,
                                       pl.ds(half * DHW + c * 16, 16)],
                                jnp.bfloat16)
                            lo, hi = plsc.unpack(
                                fb, format=plsc.PackFormat.INTERLEAVED)
                            sl0 = pl.ds(c * 32, 16)
                            sl1 = pl.ds(c * 32 + 16, 16)
                            gath_v[b, r, sl0] = gath_v[b, r, sl0] * lo
                            gath_v[b, r, sl1] = gath_v[b, r, sl1] * hi

                pltpu.sync_copy(gath_v.at[b], agg_sh.at[didx_v.at[b]],
                                add=True)

                @pl.when(gg + 2 < N_CHUNKS)
                def _pref():
                    idx_start(gg + 2, b)

        plsc.subcore_barrier()
        pltpu.sync_copy(agg_sh.at[pl.ds(row0, ROWS_PER_SUBCORE)],
                        out_hbm.at[cid, pl.ds(row0, ROWS_PER_SUBCORE)])

    return conv(h_pre, src_r, dst_r, filt)


def _final_body(p_ref, x_ref, wpost_ref, bpost_ref, g_ref, b_ref,
                w1_ref, b1_ref, w2_ref, b2_ref, o_ref):
    agg = p_ref[0] + p_ref[1]
    h = (jnp.dot(agg, wpost_ref[...], preferred_element_type=jnp.float32)
         + bpost_ref[...])
    h = h * jax.nn.sigmoid(h) + x_ref[...]
    mu = jnp.mean(h, axis=-1, keepdims=True)
    hc = h - mu
    var = jnp.mean(hc * hc, axis=-1, keepdims=True)
    f = hc * lax.rsqrt(var + 1e-5) * g_ref[...] + b_ref[...]
    f1 = (jnp.dot(f, w1_ref[...], preferred_element_type=jnp.float32)
          + b1_ref[...])
    f1 = f1 * jax.nn.sigmoid(f1)
    o_ref[...] = (jnp.dot(f1, w2_ref[...], preferred_element_type=jnp.float32)
                  + b2_ref[...] + h)


def _final(partials, x, W_post, b_post, ln2_g, ln2_b, W_ff1, b_ff1,
           W_ff2, b_ff2):
    bn = 1000
    return pl.pallas_call(
        _final_body,
        grid=(N // bn,),
        in_specs=[
            pl.BlockSpec((NC, bn, DH), lambda i: (0, i, 0)),
            pl.BlockSpec((bn, D), lambda i: (i, 0)),
            pl.BlockSpec((DH, D), lambda i: (0, 0)),
            pl.BlockSpec((1, D), lambda i: (0, 0)),
            pl.BlockSpec((1, D), lambda i: (0, 0)),
            pl.BlockSpec((1, D), lambda i: (0, 0)),
            pl.BlockSpec((D, DFF), lambda i: (0, 0)),
            pl.BlockSpec((1, DFF), lambda i: (0, 0)),
            pl.BlockSpec((DFF, D), lambda i: (0, 0)),
            pl.BlockSpec((1, D), lambda i: (0, 0)),
        ],
        out_specs=pl.BlockSpec((bn, D), lambda i: (i, 0)),
        out_shape=jax.ShapeDtypeStruct((N, D), jnp.float32),
    )(partials, x, W_post, b_post.reshape(1, D), ln2_g.reshape(1, D),
      ln2_b.reshape(1, D), W_ff1, b_ff1.reshape(1, DFF), W_ff2,
      b_ff2.reshape(1, D))


def kernel(x, edge_index, radial_basis, W_pre, b_pre, W_rf, b_rf,
           W_post, b_post, ln1_g, ln1_b, ln2_g, ln2_b,
           W_ff1, b_ff1, W_ff2, b_ff2):
    pad = E_PAD - E
    src_p = jnp.concatenate(
        [edge_index[0].astype(jnp.int32), jnp.zeros((pad,), jnp.int32)])
    dst_p = jnp.concatenate(
        [edge_index[1].astype(jnp.int32), jnp.zeros((pad,), jnp.int32)])
    rb_p = jnp.concatenate(
        [radial_basis, jnp.zeros((pad, DR), jnp.float32)])

    lcols = jnp.asarray(_LCOLS)
    hcols = jnp.asarray(_HCOLS)
    h_pre = _h_pre(x, ln1_g, ln1_b, W_pre, b_pre)
    filt = _filt(rb_p, W_rf[:, lcols], W_rf[:, hcols],
                 b_rf[lcols].reshape(1, DHW), b_rf[hcols].reshape(1, DHW))
    partials = _sc_conv(h_pre, src_p, dst_p,
                        filt.reshape(E_PAD // 2, DH))
    return _final(partials, x, W_post, b_post, ln2_g, ln2_b,
                  W_ff1, b_ff1, W_ff2, b_ff2)


# speedup vs baseline: 2.3540x; 1.1079x over previous
import dataclasses
import functools

import jax
import jax.numpy as jnp
import numpy as np
from jax import lax
from jax.experimental import pallas as pl
from jax.experimental.pallas import tpu as pltpu
from jax.experimental.pallas import tpu_sc as plsc

N = 10000
D = 128
DR = 16
DH = 128
DFF = 512
E = 320000

NC = 2
NS = 16
NW = NC * NS
CHUNK = 64
E_PER_TILE = 10240
E_PAD = NW * E_PER_TILE
N_CHUNKS = E_PER_TILE // CHUNK
GC = 8
NG = N_CHUNKS // GC
N_ACC = 10112
ROWS_PER_SUBCORE = N_ACC // NS

_LCOLS = np.concatenate([np.arange(16) + 32 * c for c in range(4)])
_HCOLS = _LCOLS + 16
DHW = DH // 2


def _pack16(lo_f32, hi_f32):
    lo = lax.convert_element_type(
        lax.bitcast_convert_type(lo_f32.astype(jnp.bfloat16), jnp.uint16),
        jnp.uint32)
    hi = lax.convert_element_type(
        lax.bitcast_convert_type(hi_f32.astype(jnp.bfloat16), jnp.uint16),
        jnp.uint32)
    return lax.bitcast_convert_type(lo | (hi << 16), jnp.int32)


def _pre_body(x_ref, g_ref, b_ref, w_ref, bias_ref, o_ref):
    x = x_ref[...]
    mu = jnp.mean(x, axis=-1, keepdims=True)
    xc = x - mu
    var = jnp.mean(xc * xc, axis=-1, keepdims=True)
    xn = xc * lax.rsqrt(var + 1e-5) * g_ref[...] + b_ref[...]
    o_ref[...] = (jnp.dot(xn, w_ref[...], preferred_element_type=jnp.float32)
                  + bias_ref[...])


def _h_pre(x, ln1_g, ln1_b, W_pre, b_pre):
    bn = 1000
    return pl.pallas_call(
        _pre_body,
        grid=(N // bn,),
        in_specs=[
            pl.BlockSpec((bn, D), lambda i: (i, 0)),
            pl.BlockSpec((1, D), lambda i: (0, 0)),
            pl.BlockSpec((1, D), lambda i: (0, 0)),
            pl.BlockSpec((D, DH), lambda i: (0, 0)),
            pl.BlockSpec((1, DH), lambda i: (0, 0)),
        ],
        out_specs=pl.BlockSpec((bn, DH), lambda i: (i, 0)),
        out_shape=jax.ShapeDtypeStruct((N, DH), jnp.float32),
    )(x, ln1_g.reshape(1, D), ln1_b.reshape(1, D), W_pre,
      b_pre.reshape(1, DH))


_FILT_BE = 2560


def _filt_body(rb_ref, wl_ref, wh_ref, bl_ref, bh_ref, o_ref):
    i = pl.program_id(0)
    rb = rb_ref[...].astype(jnp.bfloat16)
    fl = (jnp.dot(rb, wl_ref[...],
                  preferred_element_type=jnp.float32) + bl_ref[...])
    fh = (jnp.dot(rb, wh_ref[...],
                  preferred_element_type=jnp.float32) + bh_ref[...])
    rows = lax.broadcasted_iota(jnp.int32, fl.shape, 0) + i * _FILT_BE
    fl = jnp.where(rows < E, fl, 0.0)
    fh = jnp.where(rows < E, fh, 0.0)
    o_ref[...] = _pack16(fl, fh)


def _filt(rb, W_l, W_h, b_l, b_h):
    be = _FILT_BE
    return pl.pallas_call(
        _filt_body,
        grid=(E_PAD // be,),
        in_specs=[
            pl.BlockSpec((be, DR), lambda i: (i, 0)),
            pl.BlockSpec((DR, DHW), lambda i: (0, 0)),
            pl.BlockSpec((DR, DHW), lambda i: (0, 0)),
            pl.BlockSpec((1, DHW), lambda i: (0, 0)),
            pl.BlockSpec((1, DHW), lambda i: (0, 0)),
        ],
        out_specs=pl.BlockSpec((be, DHW), lambda i: (i, 0)),
        out_shape=jax.ShapeDtypeStruct((E_PAD, DHW), jnp.int32),
    )(rb, W_l, W_h, b_l, b_h)


def _sc_conv(h_pre, src_p, dst2, filt):
    mesh = plsc.VectorSubcoreMesh(core_axis_name="c", subcore_axis_name="s")
    cp = pltpu.CompilerParams(use_tc_tiling_on_sc=True)
    if "needs_layout_passes" in pltpu.CompilerParams.__dataclass_fields__:
        cp = dataclasses.replace(cp, needs_layout_passes=False)

    @functools.partial(
        pl.kernel,
        mesh=mesh,
        compiler_params=cp,
        out_type=jax.ShapeDtypeStruct((NC, N_ACC, DH), jnp.float32),
        scratch_types=[
            pltpu.VMEM((2, GC * CHUNK), jnp.int32),
            pltpu.VMEM((2, GC, CHUNK), jnp.int32),
            pltpu.VMEM((2, CHUNK, DH), jnp.float32),
            pltpu.VMEM((2, CHUNK, DHW), jnp.int32),
            pltpu.VMEM_SHARED((N_ACC, DH), jnp.float32),
            pltpu.SemaphoreType.DMA,
            pltpu.SemaphoreType.DMA,
            pltpu.SemaphoreType.DMA,
            pltpu.SemaphoreType.DMA,
            pltpu.SemaphoreType.DMA,
            pltpu.SemaphoreType.DMA,
        ],
    )
    def conv(h_hbm, src_hbm, dst_hbm, filt_hbm, out_hbm,
             sidx_v, didx_v, gath_v, filt_v, agg_sh,
             si0, si1, sg0, sg1, sf0, sf1):
        sis, sgs, sfs = (si0, si1), (sg0, sg1), (sf0, sf1)
        cid = lax.axis_index("c")
        sid = lax.axis_index("s")
        wid = sid * NC + cid
        ebase = wid * E_PER_TILE
        rbase = wid * (E_PER_TILE // CHUNK)

        def idxg_start(grp, s):
            eo = pl.ds(ebase + grp * GC * CHUNK, GC * CHUNK)
            pltpu.async_copy(src_hbm.at[eo], sidx_v.at[s], sis[s])
            ro = pl.multiple_of(rbase + grp * GC, GC)
            pltpu.async_copy(dst_hbm.at[pl.ds(ro, GC)], didx_v.at[s], sis[s])

        def idxg_wait(grp, s):
            eo = pl.ds(ebase + grp * GC * CHUNK, GC * CHUNK)
            pltpu.make_async_copy(src_hbm.at[eo], sidx_v.at[s],
                                  sis[s]).wait()
            ro = pl.multiple_of(rbase + grp * GC, GC)
            pltpu.make_async_copy(dst_hbm.at[pl.ds(ro, GC)], didx_v.at[s],
                                  sis[s]).wait()

        def gf_start(gg, s, sg, j):
            pltpu.async_copy(h_hbm.at[sidx_v.at[sg, pl.ds(j * CHUNK, CHUNK)]],
                             gath_v.at[s], sgs[s])
            fo = pl.multiple_of(ebase + gg * CHUNK, CHUNK)
            pltpu.async_copy(filt_hbm.at[pl.ds(fo, CHUNK)],
                             filt_v.at[s], sfs[s])

        def gf_wait(gg, s, sg, j):
            pltpu.make_async_copy(
                h_hbm.at[sidx_v.at[sg, pl.ds(j * CHUNK, CHUNK)]],
                gath_v.at[s], sgs[s]).wait()
            fo = pl.multiple_of(ebase + gg * CHUNK, CHUNK)
            pltpu.make_async_copy(filt_hbm.at[pl.ds(fo, CHUNK)],
                                  filt_v.at[s], sfs[s]).wait()

        idxg_start(0, 0)
        idxg_start(1, 1)

        @pl.loop(0, CHUNK)
        def _zrow(r):
            for c in range(DH // 16):
                gath_v[0, r, pl.ds(c * 16, 16)] = jnp.zeros((16,),
                                                            jnp.float32)

        row0 = sid * ROWS_PER_SUBCORE

        for j in range(ROWS_PER_SUBCORE // CHUNK):
            pltpu.sync_copy(gath_v.at[0],
                            agg_sh.at[pl.ds(row0 + j * CHUNK, CHUNK)])
        _tail = ROWS_PER_SUBCORE % CHUNK
        pltpu.sync_copy(
            gath_v.at[0, pl.ds(0, _tail)],
            agg_sh.at[pl.ds(row0 + ROWS_PER_SUBCORE - _tail, _tail)])

        idxg_wait(0, 0)
        gf_start(0, 0, 0, 0)
        plsc.subcore_barrier()

        @pl.loop(0, NG, step=2)
        def _grploop(g2):
            for gb in range(2):
                for j in range(GC):
                    grp = g2 + gb
                    gg = grp * GC + j
                    sg = gb
                    b = j % 2

                    gf_wait(gg, b, sg, j)

                    if j < GC - 1:
                        gf_start(gg + 1, 1 - b, sg, j + 1)
                    else:
                        @pl.when(grp + 1 < NG)
                        def _nxt():
                            idxg_wait(grp + 1, 1 - sg)
                            gf_start(gg + 1, 1 - b, 1 - sg, 0)

                    @pl.loop(0, CHUNK)
                    def _mul(r):
                        for c in range(DHW // 16):
                            fb = plsc.bitcast(
                                filt_v[b, r, pl.ds(c * 16, 16)],
                                jnp.bfloat16)
                            lo, hi = plsc.unpack(
                                fb, format=plsc.PackFormat.INTERLEAVED)
                            sl0 = pl.ds(c * 32, 16)
                            sl1 = pl.ds(c * 32 + 16, 16)
                            gath_v[b, r, sl0] = gath_v[b, r, sl0] * lo
                            gath_v[b, r, sl1] = gath_v[b, r, sl1] * hi

                    pltpu.sync_copy(gath_v.at[b],
                                    agg_sh.at[didx_v.at[sg, j]], add=True)

                    if j == GC - 1:
                        @pl.when(grp + 2 < NG)
                        def _pref():
                            idxg_start(grp + 2, sg)

        plsc.subcore_barrier()
        pltpu.sync_copy(agg_sh.at[pl.ds(row0, ROWS_PER_SUBCORE)],
                        out_hbm.at[cid, pl.ds(row0, ROWS_PER_SUBCORE)])

    return conv(h_pre, src_p, dst2, filt)


def _final_body(p_ref, x_ref, wpost_ref, bpost_ref, g_ref, b_ref,
                w1_ref, b1_ref, w2_ref, b2_ref, o_ref):
    agg = p_ref[0] + p_ref[1]
    h = (jnp.dot(agg, wpost_ref[...], preferred_element_type=jnp.float32)
         + bpost_ref[...])
    h = h * jax.nn.sigmoid(h) + x_ref[...]
    mu = jnp.mean(h, axis=-1, keepdims=True)
    hc = h - mu
    var = jnp.mean(hc * hc, axis=-1, keepdims=True)
    f = hc * lax.rsqrt(var + 1e-5) * g_ref[...] + b_ref[...]
    f1 = (jnp.dot(f, w1_ref[...], preferred_element_type=jnp.float32)
          + b1_ref[...])
    f1 = f1 * jax.nn.sigmoid(f1)
    o_ref[...] = (jnp.dot(f1, w2_ref[...], preferred_element_type=jnp.float32)
                  + b2_ref[...] + h)


def _final(partials, x, W_post, b_post, ln2_g, ln2_b, W_ff1, b_ff1,
           W_ff2, b_ff2):
    bn = 1000
    return pl.pallas_call(
        _final_body,
        grid=(N // bn,),
        in_specs=[
            pl.BlockSpec((NC, bn, DH), lambda i: (0, i, 0)),
            pl.BlockSpec((bn, D), lambda i: (i, 0)),
            pl.BlockSpec((DH, D), lambda i: (0, 0)),
            pl.BlockSpec((1, D), lambda i: (0, 0)),
            pl.BlockSpec((1, D), lambda i: (0, 0)),
            pl.BlockSpec((1, D), lambda i: (0, 0)),
            pl.BlockSpec((D, DFF), lambda i: (0, 0)),
            pl.BlockSpec((1, DFF), lambda i: (0, 0)),
            pl.BlockSpec((DFF, D), lambda i: (0, 0)),
            pl.BlockSpec((1, D), lambda i: (0, 0)),
        ],
        out_specs=pl.BlockSpec((bn, D), lambda i: (i, 0)),
        out_shape=jax.ShapeDtypeStruct((N, D), jnp.float32),
    )(partials, x, W_post, b_post.reshape(1, D), ln2_g.reshape(1, D),
      ln2_b.reshape(1, D), W_ff1, b_ff1.reshape(1, DFF), W_ff2,
      b_ff2.reshape(1, D))


def kernel(x, edge_index, radial_basis, W_pre, b_pre, W_rf, b_rf,
           W_post, b_post, ln1_g, ln1_b, ln2_g, ln2_b,
           W_ff1, b_ff1, W_ff2, b_ff2):
    pad = E_PAD - E
    src_p = jnp.concatenate(
        [edge_index[0].astype(jnp.int32), jnp.zeros((pad,), jnp.int32)])
    dst_p = jnp.concatenate(
        [edge_index[1].astype(jnp.int32), jnp.zeros((pad,), jnp.int32)])

    lcols = jnp.asarray(_LCOLS)
    hcols = jnp.asarray(_HCOLS)
    h_pre = _h_pre(x, ln1_g, ln1_b, W_pre, b_pre)
    filt = _filt(radial_basis,
                 W_rf[:, lcols].astype(jnp.bfloat16),
                 W_rf[:, hcols].astype(jnp.bfloat16),
                 b_rf[lcols].reshape(1, DHW), b_rf[hcols].reshape(1, DHW))
    partials = _sc_conv(h_pre, src_p,
                        dst_p.reshape(E_PAD // CHUNK, CHUNK), filt)
    return _final(partials, x, W_post, b_post, ln2_g, ln2_b,
                  W_ff1, b_ff1, W_ff2, b_ff2)


# speedup vs baseline: 2.4352x; 1.0345x over previous
import dataclasses
import functools

import jax
import jax.numpy as jnp
import numpy as np
from jax import lax
from jax.experimental import pallas as pl
from jax.experimental.pallas import tpu as pltpu
from jax.experimental.pallas import tpu_sc as plsc

N = 10000
D = 128
DR = 16
DH = 128
DFF = 512
E = 320000

NC = 2
NS = 16
NW = NC * NS
CHUNK = 64
E_PER_TILE = 10240
E_PAD = NW * E_PER_TILE
N_CHUNKS = E_PER_TILE // CHUNK
GC = 8
NG = N_CHUNKS // GC
N_ACC = 10112
ROWS_PER_SUBCORE = N_ACC // NS

_LCOLS = np.concatenate([np.arange(16) + 32 * c for c in range(4)])
_HCOLS = _LCOLS + 16
DHW = DH // 2


def _pack16(lo_f32, hi_f32):
    lo = lax.convert_element_type(
        lax.bitcast_convert_type(lo_f32.astype(jnp.bfloat16), jnp.uint16),
        jnp.uint32)
    hi = lax.convert_element_type(
        lax.bitcast_convert_type(hi_f32.astype(jnp.bfloat16), jnp.uint16),
        jnp.uint32)
    return lax.bitcast_convert_type(lo | (hi << 16), jnp.int32)


def _pre_body(x_ref, g_ref, b_ref, w_ref, bias_ref, o_ref):
    x = x_ref[...]
    mu = jnp.mean(x, axis=-1, keepdims=True)
    xc = x - mu
    var = jnp.mean(xc * xc, axis=-1, keepdims=True)
    xn = xc * lax.rsqrt(var + 1e-5) * g_ref[...] + b_ref[...]
    o_ref[...] = (jnp.dot(xn, w_ref[...], preferred_element_type=jnp.float32)
                  + bias_ref[...])


def _h_pre(x, ln1_g, ln1_b, W_pre, b_pre):
    bn = 1000
    return pl.pallas_call(
        _pre_body,
        grid=(N // bn,),
        in_specs=[
            pl.BlockSpec((bn, D), lambda i: (i, 0)),
            pl.BlockSpec((1, D), lambda i: (0, 0)),
            pl.BlockSpec((1, D), lambda i: (0, 0)),
            pl.BlockSpec((D, DH), lambda i: (0, 0)),
            pl.BlockSpec((1, DH), lambda i: (0, 0)),
        ],
        out_specs=pl.BlockSpec((bn, DH), lambda i: (i, 0)),
        out_shape=jax.ShapeDtypeStruct((N, DH), jnp.float32),
    )(x, ln1_g.reshape(1, D), ln1_b.reshape(1, D), W_pre,
      b_pre.reshape(1, DH))


_FILT_BE = 2048
E8 = E // 8
E8_PAD = E_PAD // 8


def _filt_body(rb_ref, w_ref, bias_ref, o_ref):
    i = pl.program_id(0)
    p = (jnp.dot(rb_ref[...].astype(jnp.bfloat16), w_ref[...],
                 preferred_element_type=jnp.float32) + bias_ref[...])
    rows = lax.broadcasted_iota(jnp.int32, p.shape, 0) + i * _FILT_BE
    p = jnp.where(rows < E8, p, 0.0)
    o_ref[...] = jnp.concatenate(
        [_pack16(p[:, 128 * d:128 * d + 64], p[:, 128 * d + 64:128 * d + 128])
         for d in range(8)], axis=1)


def _filt(rb8p, W2, b2):
    be = _FILT_BE
    return pl.pallas_call(
        _filt_body,
        grid=(E8_PAD // be,),
        in_specs=[
            pl.BlockSpec((be, 128), lambda i: (i, 0)),
            pl.BlockSpec((128, 1024), lambda i: (0, 0)),
            pl.BlockSpec((1, 1024), lambda i: (0, 0)),
        ],
        out_specs=pl.BlockSpec((be, 512), lambda i: (i, 0)),
        out_shape=jax.ShapeDtypeStruct((E8_PAD, 512), jnp.int32),
    )(rb8p, W2, b2)


def _sc_conv(h_pre, src_p, dst2, filt):
    mesh = plsc.VectorSubcoreMesh(core_axis_name="c", subcore_axis_name="s")
    cp = pltpu.CompilerParams(use_tc_tiling_on_sc=True)
    if "needs_layout_passes" in pltpu.CompilerParams.__dataclass_fields__:
        cp = dataclasses.replace(cp, needs_layout_passes=False)

    @functools.partial(
        pl.kernel,
        mesh=mesh,
        compiler_params=cp,
        out_type=jax.ShapeDtypeStruct((NC, N_ACC, DH), jnp.float32),
        scratch_types=[
            pltpu.VMEM((2, GC * CHUNK), jnp.int32),
            pltpu.VMEM((2, GC, CHUNK), jnp.int32),
            pltpu.VMEM((2, CHUNK, DH), jnp.float32),
            pltpu.VMEM((2, CHUNK // 8, 512), jnp.int32),
            pltpu.VMEM_SHARED((N_ACC, DH), jnp.float32),
            pltpu.SemaphoreType.DMA,
            pltpu.SemaphoreType.DMA,
            pltpu.SemaphoreType.DMA,
            pltpu.SemaphoreType.DMA,
            pltpu.SemaphoreType.DMA,
            pltpu.SemaphoreType.DMA,
        ],
    )
    def conv(h_hbm, src_hbm, dst_hbm, filt_hbm, out_hbm,
             sidx_v, didx_v, gath_v, filt_v, agg_sh,
             si0, si1, sg0, sg1, sf0, sf1):
        sis, sgs, sfs = (si0, si1), (sg0, sg1), (sf0, sf1)
        cid = lax.axis_index("c")
        sid = lax.axis_index("s")
        wid = sid * NC + cid
        ebase = wid * E_PER_TILE
        rbase = wid * (E_PER_TILE // CHUNK)
        fbase = wid * (E_PER_TILE // 8)

        def idxg_start(grp, s):
            eo = pl.ds(ebase + grp * GC * CHUNK, GC * CHUNK)
            pltpu.async_copy(src_hbm.at[eo], sidx_v.at[s], sis[s])
            ro = pl.multiple_of(rbase + grp * GC, GC)
            pltpu.async_copy(dst_hbm.at[pl.ds(ro, GC)], didx_v.at[s], sis[s])

        def idxg_wait(grp, s):
            eo = pl.ds(ebase + grp * GC * CHUNK, GC * CHUNK)
            pltpu.make_async_copy(src_hbm.at[eo], sidx_v.at[s],
                                  sis[s]).wait()
            ro = pl.multiple_of(rbase + grp * GC, GC)
            pltpu.make_async_copy(dst_hbm.at[pl.ds(ro, GC)], didx_v.at[s],
                                  sis[s]).wait()

        def gf_start(gg, s, sg, j):
            pltpu.async_copy(h_hbm.at[sidx_v.at[sg, pl.ds(j * CHUNK, CHUNK)]],
                             gath_v.at[s], sgs[s])
            fo = pl.multiple_of(fbase + gg * (CHUNK // 8), CHUNK // 8)
            pltpu.async_copy(filt_hbm.at[pl.ds(fo, CHUNK // 8)],
                             filt_v.at[s], sfs[s])

        def gf_wait(gg, s, sg, j):
            pltpu.make_async_copy(
                h_hbm.at[sidx_v.at[sg, pl.ds(j * CHUNK, CHUNK)]],
                gath_v.at[s], sgs[s]).wait()
            fo = pl.multiple_of(fbase + gg * (CHUNK // 8), CHUNK // 8)
            pltpu.make_async_copy(filt_hbm.at[pl.ds(fo, CHUNK // 8)],
                                  filt_v.at[s], sfs[s]).wait()

        idxg_start(0, 0)
        idxg_start(1, 1)

        @pl.loop(0, CHUNK)
        def _zrow(r):
            for c in range(DH // 16):
                gath_v[0, r, pl.ds(c * 16, 16)] = jnp.zeros((16,),
                                                            jnp.float32)

        row0 = sid * ROWS_PER_SUBCORE

        for j in range(ROWS_PER_SUBCORE // CHUNK):
            pltpu.sync_copy(gath_v.at[0],
                            agg_sh.at[pl.ds(row0 + j * CHUNK, CHUNK)])
        _tail = ROWS_PER_SUBCORE % CHUNK
        pltpu.sync_copy(
            gath_v.at[0, pl.ds(0, _tail)],
            agg_sh.at[pl.ds(row0 + ROWS_PER_SUBCORE - _tail, _tail)])

        idxg_wait(0, 0)
        gf_start(0, 0, 0, 0)
        plsc.subcore_barrier()

        @pl.loop(0, NG, step=2)
        def _grploop(g2):
            for gb in range(2):
                for j in range(GC):
                    grp = g2 + gb
                    gg = grp * GC + j
                    sg = gb
                    b = j % 2

                    gf_wait(gg, b, sg, j)

                    if j < GC - 1:
                        gf_start(gg + 1, 1 - b, sg, j + 1)
                    else:
                        @pl.when(grp + 1 < NG)
                        def _nxt():
                            idxg_wait(grp + 1, 1 - sg)
                            gf_start(gg + 1, 1 - b, 1 - sg, 0)

                    @pl.loop(0, CHUNK // 8)
                    def _mul(rp):
                        for d in range(8):
                            r = rp * 8 + d
                            for c in range(DHW // 16):
                                fb = plsc.bitcast(
                                    filt_v[b, rp,
                                           pl.ds(d * 64 + c * 16, 16)],
                                    jnp.bfloat16)
                                lo, hi = plsc.unpack(
                                    fb, format=plsc.PackFormat.INTERLEAVED)
                                sl0 = pl.ds(c * 32, 16)
                                sl1 = pl.ds(c * 32 + 16, 16)
                                gath_v[b, r, sl0] = gath_v[b, r, sl0] * lo
                                gath_v[b, r, sl1] = gath_v[b, r, sl1] * hi

                    pltpu.sync_copy(gath_v.at[b],
                                    agg_sh.at[didx_v.at[sg, j]], add=True)

                    if j == GC - 1:
                        @pl.when(grp + 2 < NG)
                        def _pref():
                            idxg_start(grp + 2, sg)

        plsc.subcore_barrier()
        pltpu.sync_copy(agg_sh.at[pl.ds(row0, ROWS_PER_SUBCORE)],
                        out_hbm.at[cid, pl.ds(row0, ROWS_PER_SUBCORE)])

    return conv(h_pre, src_p, dst2, filt)


def _final_body(p_ref, x_ref, wpost_ref, bpost_ref, g_ref, b_ref,
                w1_ref, b1_ref, w2_ref, b2_ref, o_ref):
    agg = p_ref[0] + p_ref[1]
    h = (jnp.dot(agg, wpost_ref[...], preferred_element_type=jnp.float32)
         + bpost_ref[...])
    h = h * jax.nn.sigmoid(h) + x_ref[...]
    mu = jnp.mean(h, axis=-1, keepdims=True)
    hc = h - mu
    var = jnp.mean(hc * hc, axis=-1, keepdims=True)
    f = hc * lax.rsqrt(var + 1e-5) * g_ref[...] + b_ref[...]
    f1 = (jnp.dot(f, w1_ref[...], preferred_element_type=jnp.float32)
          + b1_ref[...])
    f1 = f1 * jax.nn.sigmoid(f1)
    o_ref[...] = (jnp.dot(f1, w2_ref[...], preferred_element_type=jnp.float32)
                  + b2_ref[...] + h)


def _final(partials, x, W_post, b_post, ln2_g, ln2_b, W_ff1, b_ff1,
           W_ff2, b_ff2):
    bn = 1000
    return pl.pallas_call(
        _final_body,
        grid=(N // bn,),
        in_specs=[
            pl.BlockSpec((NC, bn, DH), lambda i: (0, i, 0)),
            pl.BlockSpec((bn, D), lambda i: (i, 0)),
            pl.BlockSpec((DH, D), lambda i: (0, 0)),
            pl.BlockSpec((1, D), lambda i: (0, 0)),
            pl.BlockSpec((1, D), lambda i: (0, 0)),
            pl.BlockSpec((1, D), lambda i: (0, 0)),
            pl.BlockSpec((D, DFF), lambda i: (0, 0)),
            pl.BlockSpec((1, DFF), lambda i: (0, 0)),
            pl.BlockSpec((DFF, D), lambda i: (0, 0)),
            pl.BlockSpec((1, D), lambda i: (0, 0)),
        ],
        out_specs=pl.BlockSpec((bn, D), lambda i: (i, 0)),
        out_shape=jax.ShapeDtypeStruct((N, D), jnp.float32),
    )(partials, x, W_post, b_post.reshape(1, D), ln2_g.reshape(1, D),
      ln2_b.reshape(1, D), W_ff1, b_ff1.reshape(1, DFF), W_ff2,
      b_ff2.reshape(1, D))


def kernel(x, edge_index, radial_basis, W_pre, b_pre, W_rf, b_rf,
           W_post, b_post, ln1_g, ln1_b, ln2_g, ln2_b,
           W_ff1, b_ff1, W_ff2, b_ff2):
    pad = E_PAD - E
    src_p = jnp.concatenate(
        [edge_index[0].astype(jnp.int32), jnp.zeros((pad,), jnp.int32)])
    dst_p = jnp.concatenate(
        [edge_index[1].astype(jnp.int32), jnp.zeros((pad,), jnp.int32)])

    lcols = jnp.asarray(_LCOLS)
    hcols = jnp.asarray(_HCOLS)
    h_pre = _h_pre(x, ln1_g, ln1_b, W_pre, b_pre)
    wlh = jnp.concatenate([W_rf[:, lcols], W_rf[:, hcols]], axis=1)
    W2 = jax.scipy.linalg.block_diag(*([wlh] * 8)).astype(jnp.bfloat16)
    b2 = jnp.tile(jnp.concatenate([b_rf[lcols], b_rf[hcols]]), 8)
    rb8p = jnp.concatenate(
        [radial_basis.reshape(E8, 128),
         jnp.zeros((E8_PAD - E8, 128), jnp.float32)])
    filt = _filt(rb8p, W2, b2.reshape(1, 1024))
    partials = _sc_conv(h_pre, src_p,
                        dst_p.reshape(E_PAD // CHUNK, CHUNK), filt)
    return _final(partials, x, W_post, b_post, ln2_g, ln2_b,
                  W_ff1, b_ff1, W_ff2, b_ff2)
